# Initial kernel scaffold; baseline (speedup 1.0000x reference)
import jax
import jax.numpy as jnp
from jax.experimental import pallas as pl


def kernel(input):
    raise NotImplementedError("write your pallas kernel here")


# speedup vs baseline: 1.7161x; 1.7161x over previous
import functools

import jax
import jax.numpy as jnp
from jax import lax
from jax.experimental import pallas as pl
from jax.experimental.pallas import tpu as pltpu
from jax.experimental.pallas import tpu_sc as plsc

B = 4096
NROW = 72
D = 256
NC = 2
NS = 16
NW = NC * NS
PER_W = B // NW
CB = 32
NCHUNK = PER_W // CB

_mesh = plsc.VectorSubcoreMesh(core_axis_name="c", subcore_axis_name="s")


@functools.partial(
    pl.kernel,
    out_type=jax.ShapeDtypeStruct((B, 6, D), jnp.float32),
    mesh=_mesh,
    scratch_types=[
        pltpu.VMEM((CB, 6, D), jnp.float32),
    ],
    compiler_params=pltpu.CompilerParams(use_tc_tiling_on_sc=False),
)
def _select_hands(x_hbm, out_hbm, buf):
    wid = lax.axis_index("s") * NC + lax.axis_index("c")
    for g in range(NCHUNK):
        base = wid * PER_W + g * CB
        pltpu.sync_copy(
            x_hbm.at[pl.ds(base, CB), pl.ds(27, 3)], buf.at[:, pl.ds(0, 3)]
        )
        pltpu.sync_copy(
            x_hbm.at[pl.ds(base, CB), pl.ds(39, 3)], buf.at[:, pl.ds(3, 3)]
        )
        pltpu.sync_copy(buf, out_hbm.at[pl.ds(base, CB)])


def kernel(input):
    return _select_hands(input)
